# baseline (speedup 1.0000x reference)
import jax
import jax.numpy as jnp
from jax.experimental import pallas as pl

N = 1024
B = 2
HID = 1024
LAT = 512
NHEADS = 4
NHID = 64
ALPHA = 0.2
_NEG_LOG2E = -1.4426950408889634


def _elu(v):
    return jnp.where(v > 0, v, jnp.exp(v) - 1.0)


def _deepim_body(x_ref, adj_ref, w1_ref, b1_ref, w2_ref, b2_ref, w3_ref, b3_ref,
                 d1_ref, db1_ref, d2_ref, db2_ref, d3_ref, db3_ref, d4_ref, db4_ref,
                 gw_ref, ga_ref, ow_ref, oa_ref, xhat_ref, yhat_ref):
    x = x_ref[...]
    adj = adj_ref[...]

    def _mm(a, w_ref):
        return jnp.dot(a.astype(jnp.bfloat16), w_ref[...],
                       preferred_element_type=jnp.float32)

    h = jax.nn.relu(_mm(x, w1_ref) + b1_ref[...])
    h = jax.nn.relu(_mm(h, w2_ref) + b2_ref[...])
    h = jax.nn.relu(_mm(h, w2_ref) + b2_ref[...])
    z = _mm(h, w3_ref) + b3_ref[...]
    h = jax.nn.relu(_mm(z, d1_ref) + db1_ref[...])
    h = jax.nn.relu(_mm(h, d2_ref) + db2_ref[...])
    h = jax.nn.relu(_mm(h, d3_ref) + db3_ref[...])
    x_hat = jax.nn.sigmoid(_mm(h, d4_ref) + db4_ref[...])
    xhat_ref[...] = x_hat

    gw = gw_ref[...]
    ga = ga_ref[...]
    ow = ow_ref[...]
    oa = oa_ref[...]

    for b in range(B):
        xn_row = x_hat[b:b + 1, :]
        xn_col = jnp.transpose(xn_row)

        blocks = []
        for i in range(NHEADS):
            gw_i = gw[i:i + 1, :]
            c1 = jnp.sum(gw_i * ga[i:i + 1, :NHID], axis=1, keepdims=True)
            c2 = jnp.sum(gw_i * ga[i:i + 1, NHID:], axis=1, keepdims=True)
            m1 = xn_col * (c1 * _NEG_LOG2E)
            n1 = xn_row * (c2 * _NEG_LOG2E)
            m2 = m1 * ALPHA
            n2 = n1 * ALPHA
            e = jnp.exp2(jnp.minimum(m1 + n1, m2 + n2)) * adj
            den = jnp.sum(e, axis=1, keepdims=True)
            num = jnp.sum(e * xn_row, axis=1, keepdims=True)
            p = num / den
            blocks.append(_elu(_elu(p * gw_i)))
        xh = jnp.concatenate(blocks, axis=1)

        h2 = jnp.dot(xh, ow, preferred_element_type=jnp.float32)
        h2_row = jnp.transpose(h2)
        m1 = h2 * (oa[0:1, 0:1] * _NEG_LOG2E)
        n1 = h2_row * (oa[0:1, 1:2] * _NEG_LOG2E)
        m2 = m1 * ALPHA
        n2 = n1 * ALPHA
        e2 = jnp.exp2(jnp.minimum(m1 + n1, m2 + n2)) * adj
        den2 = jnp.sum(e2, axis=1, keepdims=True)
        num2 = jnp.sum(e2 * h2_row, axis=1, keepdims=True)
        y = _elu(num2 / den2)
        yhat_ref[b:b + 1, :] = jnp.transpose(y)


def kernel(x, adj, enc_w1, enc_b1, enc_w2, enc_b2, enc_w3, enc_b3,
           dec_w1, dec_b1, dec_w2, dec_b2, dec_w3, dec_b3, dec_w4, dec_b4,
           gat_W, gat_a, out_W, out_a):
    bf = jnp.bfloat16
    args = (
        x, adj,
        enc_w1.astype(bf), enc_b1.reshape(1, HID), enc_w2.astype(bf), enc_b2.reshape(1, HID),
        enc_w3.astype(bf), enc_b3.reshape(1, LAT),
        dec_w1.astype(bf), dec_b1.reshape(1, LAT), dec_w2.astype(bf), dec_b2.reshape(1, HID),
        dec_w3.astype(bf), dec_b3.reshape(1, HID), dec_w4.astype(bf), dec_b4.reshape(1, N),
        gat_W.reshape(NHEADS, NHID), gat_a.reshape(NHEADS, 2 * NHID),
        out_W, out_a,
    )
    x_hat, y_hat = pl.pallas_call(
        _deepim_body,
        out_shape=(
            jax.ShapeDtypeStruct((B, N), jnp.float32),
            jax.ShapeDtypeStruct((B, N), jnp.float32),
        ),
    )(*args)
    return x_hat, y_hat


# speedup vs baseline: 1.5615x; 1.5615x over previous
import jax
import jax.numpy as jnp
from jax.experimental import pallas as pl

N = 1024
B = 2
HID = 1024
LAT = 512
NHEADS = 4
NHID = 64
ALPHA = 0.2
_NEG_LOG2E = -1.4426950408889634


def _elu(v):
    return jnp.where(v > 0, v, jnp.exp(v) - 1.0)


def _deepim_body(x_ref, adj_ref, w1_ref, b1_ref, w2_ref, b2_ref, w3_ref, b3_ref,
                 d1_ref, db1_ref, d2_ref, db2_ref, d3_ref, db3_ref, d4_ref, db4_ref,
                 gw_ref, ga_ref, ow_ref, oa_ref, xhat_ref, yhat_ref):
    x = x_ref[...]
    adj = adj_ref[...]

    def _mm(a, w_ref):
        return jnp.dot(a, w_ref[...], preferred_element_type=jnp.float32)

    h = jax.nn.relu(_mm(x, w1_ref) + b1_ref[...])
    h = jax.nn.relu(_mm(h, w2_ref) + b2_ref[...])
    h = jax.nn.relu(_mm(h, w2_ref) + b2_ref[...])
    z = _mm(h, w3_ref) + b3_ref[...]
    h = jax.nn.relu(_mm(z, d1_ref) + db1_ref[...])
    h = jax.nn.relu(_mm(h, d2_ref) + db2_ref[...])
    h = jax.nn.relu(_mm(h, d3_ref) + db3_ref[...])
    x_hat = jax.nn.sigmoid(_mm(h, d4_ref) + db4_ref[...])
    xhat_ref[...] = x_hat

    gw = gw_ref[...]
    ga = ga_ref[...]
    ow = ow_ref[...]
    oa = oa_ref[...]

    for b in range(B):
        xn_row = x_hat[b:b + 1, :]
        xn_col = jnp.transpose(xn_row)

        blocks = []
        for i in range(NHEADS):
            gw_i = gw[i:i + 1, :]
            c1 = jnp.sum(gw_i * ga[i:i + 1, :NHID], axis=1, keepdims=True)
            c2 = jnp.sum(gw_i * ga[i:i + 1, NHID:], axis=1, keepdims=True)
            m1 = xn_col * (c1 * _NEG_LOG2E)
            n1 = xn_row * (c2 * _NEG_LOG2E)
            sp = m1 + n1
            e = jnp.exp2(jnp.minimum(sp, ALPHA * sp)) * adj
            den = jnp.sum(e, axis=1, keepdims=True)
            num = jnp.sum(e * xn_row, axis=1, keepdims=True)
            p = num / den
            blocks.append(_elu(_elu(p * gw_i)))
        xh = jnp.concatenate(blocks, axis=1)

        h2 = jnp.dot(xh, ow, preferred_element_type=jnp.float32)
        h2_row = jnp.transpose(h2)
        m1 = h2 * (oa[0:1, 0:1] * _NEG_LOG2E)
        n1 = h2_row * (oa[0:1, 1:2] * _NEG_LOG2E)
        sp = m1 + n1
        e2 = jnp.exp2(jnp.minimum(sp, ALPHA * sp)) * adj
        den2 = jnp.sum(e2, axis=1, keepdims=True)
        num2 = jnp.sum(e2 * h2_row, axis=1, keepdims=True)
        y = _elu(num2 / den2)
        yhat_ref[b:b + 1, :] = jnp.transpose(y)


def kernel(x, adj, enc_w1, enc_b1, enc_w2, enc_b2, enc_w3, enc_b3,
           dec_w1, dec_b1, dec_w2, dec_b2, dec_w3, dec_b3, dec_w4, dec_b4,
           gat_W, gat_a, out_W, out_a):
    args = (
        x, adj,
        enc_w1, enc_b1.reshape(1, HID), enc_w2, enc_b2.reshape(1, HID),
        enc_w3, enc_b3.reshape(1, LAT),
        dec_w1, dec_b1.reshape(1, LAT), dec_w2, dec_b2.reshape(1, HID),
        dec_w3, dec_b3.reshape(1, HID), dec_w4, dec_b4.reshape(1, N),
        gat_W.reshape(NHEADS, NHID), gat_a.reshape(NHEADS, 2 * NHID),
        out_W, out_a,
    )
    x_hat, y_hat = pl.pallas_call(
        _deepim_body,
        out_shape=(
            jax.ShapeDtypeStruct((B, N), jnp.float32),
            jax.ShapeDtypeStruct((B, N), jnp.float32),
        ),
    )(*args)
    return x_hat, y_hat
